# baseline (speedup 1.0000x reference)
import functools

import jax
import jax.numpy as jnp
from jax import lax
from jax.experimental import pallas as pl
from jax.experimental.pallas import tpu as pltpu
from jax.experimental.pallas import tpu_sc as plsc

KC = 8192
DD = 256
NN = 16384
BETA = 0.25

TN = 2048
TK = 4096
NT = NN // TN
NK = KC // TK
NR8 = TK // 8


def _dist_body(z_ref, cb_ref, wn_out, k_out, loss_out,
               zn2_s, w2_s, m8_s, b8_s, loss_sm):
    t = pl.program_id(0)
    kb = pl.program_id(1)

    @pl.when(t == 0)
    def _():
        wb = cb_ref[pl.ds(kb * TK, TK), :]
        nrm = jnp.sqrt(jnp.sum(wb * wb, axis=1, keepdims=True))
        wn = wb / jnp.clip(nrm, 1e-12, None)
        wn_out[pl.ds(kb * TK, TK), :] = wn
        w2_s[pl.ds(kb * TK, TK), :] = jnp.sum(wn * wn, axis=1, keepdims=True)

    @pl.when(kb == 0)
    def _():
        zb = z_ref[...]
        nrm = jnp.sqrt(jnp.sum(zb * zb, axis=1, keepdims=True))
        zn = zb / jnp.clip(nrm, 1e-12, None)
        zn2_s[...] = -2.0 * zn
        m8_s[...] = jnp.full((8, TN), jnp.inf, jnp.float32)
        b8_s[...] = jnp.zeros((8, TN), jnp.int32)
        part = jnp.sum(zn * zn)
        loss_sm[0] = jnp.where(t == 0, part, loss_sm[0] + part)

    wn = wn_out[pl.ds(kb * TK, TK), :]
    s = lax.dot_general(wn, zn2_s[...], (((1,), (1,)), ((), ())),
                        preferred_element_type=jnp.float32)
    d = s + w2_s[pl.ds(kb * TK, TK), :]
    d3 = d.reshape(NR8, 8, TN)
    m8 = m8_s[...]
    b8 = b8_s[...]
    for r in range(NR8):
        v = d3[r]
        pred = v < m8
        m8 = jnp.where(pred, v, m8)
        b8 = jnp.where(pred, jnp.int32(kb * NR8 + r), b8)
    m8_s[...] = m8
    b8_s[...] = b8

    @pl.when(kb == NK - 1)
    def _():
        cidx = b8 * 8 + lax.broadcasted_iota(jnp.int32, (8, TN), 0)
        mwin = jnp.min(m8, axis=0, keepdims=True)
        cand = jnp.where(m8 == mwin, cidx, KC)
        k_out[0, 0, :] = jnp.min(cand, axis=0)
        loss_sm[0] = loss_sm[0] + jnp.sum(mwin)

        @pl.when(t == NT - 1)
        def _():
            loss_out[...] = (BETA / float(NN * DD)) * loss_sm[0] * jnp.ones(
                (1, 1), jnp.float32)


_dist_call = pl.pallas_call(
    _dist_body,
    grid=(NT, NK),
    in_specs=[
        pl.BlockSpec((TN, DD), lambda t, kb: (t, 0)),
        pl.BlockSpec((KC, DD), lambda t, kb: (0, 0)),
    ],
    out_specs=[
        pl.BlockSpec((KC, DD), lambda t, kb: (0, 0)),
        pl.BlockSpec((1, 1, TN), lambda t, kb: (t, 0, 0)),
        pl.BlockSpec((1, 1), lambda t, kb: (0, 0)),
    ],
    out_shape=[
        jax.ShapeDtypeStruct((KC, DD), jnp.float32),
        jax.ShapeDtypeStruct((NT, 1, TN), jnp.int32),
        jax.ShapeDtypeStruct((1, 1), jnp.float32),
    ],
    scratch_shapes=[
        pltpu.VMEM((TN, DD), jnp.float32),
        pltpu.VMEM((KC, 1), jnp.float32),
        pltpu.VMEM((8, TN), jnp.float32),
        pltpu.VMEM((8, TN), jnp.int32),
        pltpu.SMEM((1,), jnp.float32),
    ],
)


CH = 128
GPW0 = 384
GPW1 = 640
NCH0 = GPW0 // CH
NCH1 = GPW1 // CH
SPT = NN // 16
NSC = SPT // CH
TSL = KC // 16


def _sc_gather_body(wn_hbm, k_hbm, zq_hbm, util_hbm,
                    idx_v, rows_v, sidx_v, zero_v, ones_v, cnt_v, util_v,
                    acc2_v, table_sh, acc_sh, g0, g1, st0, st1):
    c = lax.axis_index("c")
    s = lax.axis_index("s")

    def pipeline(base, nch):
        pltpu.sync_copy(k_hbm.at[pl.ds(base, nch * CH)],
                        idx_v.at[pl.ds(0, nch * CH)])
        gs = [g0, g1]
        ss = [st0, st1]
        gh = {}
        sh = {}
        for ch in range(nch):
            b = ch % 2
            if ch >= 2:
                sh[ch - 2].wait()
            gh[ch] = pltpu.async_copy(
                wn_hbm.at[idx_v.at[pl.ds(ch * CH, CH)]],
                rows_v.at[b], gs[b])
            if ch >= 1:
                gh[ch - 1].wait()
                sh[ch - 1] = pltpu.async_copy(
                    rows_v.at[1 - b],
                    zq_hbm.at[pl.ds(base + (ch - 1) * CH, CH)], ss[1 - b])
        gh[nch - 1].wait()
        sh[nch - 1] = pltpu.async_copy(
            rows_v.at[(nch - 1) % 2],
            zq_hbm.at[pl.ds(base + (nch - 1) * CH, CH)], ss[(nch - 1) % 2])
        if nch >= 2:
            sh[nch - 2].wait()
        sh[nch - 1].wait()

    @pl.when(c == 0)
    def _():
        for i in range(TSL // 16):
            zero_v[pl.ds(i * 16, 16)] = jnp.zeros((16,), jnp.float32)
        pltpu.sync_copy(zero_v, table_sh.at[pl.ds(s * TSL, TSL)])
        plsc.subcore_barrier()

        pipeline(s * GPW0, NCH0)

        for i in range(CH // 16):
            ones_v[pl.ds(i * 16, 16)] = jnp.ones((16,), jnp.float32)
        for j in range(NSC):
            pltpu.sync_copy(k_hbm.at[pl.ds(s * SPT + j * CH, CH)],
                            sidx_v.at[j])
        for j in range(NSC):
            pltpu.sync_copy(ones_v, table_sh.at[sidx_v.at[j]], add=True)
        plsc.subcore_barrier()

        @pl.when(s == 0)
        def _():
            def outer(q, tot0):
                pltpu.sync_copy(table_sh.at[pl.ds(q * TSL, TSL)],
                                cnt_v.at[pl.ds(0, TSL)])

                def body(i, acc):
                    v = cnt_v[pl.ds(i * 16, 16)]
                    return acc + jnp.where(v > 0.0, 1.0, 0.0)

                return tot0 + lax.fori_loop(0, TSL // 16, body,
                                            jnp.zeros((16,), jnp.float32))

            acc = outer(0, jnp.zeros((16,), jnp.float32))
            for q in range(1, 16):
                acc = outer(q, acc)
            tot = acc[0]
            for i in range(1, 16):
                tot = tot + acc[i]
            util_v[...] = jnp.full((16,), tot * (1.0 / float(KC)),
                                   jnp.float32)
            pltpu.sync_copy(util_v, util_hbm)

    @pl.when(c == 1)
    def _():
        pipeline(16 * GPW0 + s * GPW1, NCH1)


@functools.cache
def _make_sc_gather():
    mesh = plsc.VectorSubcoreMesh(core_axis_name="c", subcore_axis_name="s")
    return functools.partial(
        pl.kernel, mesh=mesh,
        out_type=[
            jax.ShapeDtypeStruct((NN, DD), jnp.float32),
            jax.ShapeDtypeStruct((16,), jnp.float32),
        ],
        scratch_types=[
            pltpu.VMEM((GPW1,), jnp.int32),
            pltpu.VMEM((2, CH, DD), jnp.float32),
            pltpu.VMEM((NSC, CH), jnp.int32),
            pltpu.VMEM((TSL,), jnp.float32),
            pltpu.VMEM((CH,), jnp.float32),
            pltpu.VMEM((KC // 16,), jnp.float32),
            pltpu.VMEM((16,), jnp.float32),
            pltpu.VMEM((16, 16), jnp.float32),
            pltpu.VMEM_SHARED((KC,), jnp.float32),
            pltpu.VMEM_SHARED((16, 16), jnp.float32),
            pltpu.SemaphoreType.DMA,
            pltpu.SemaphoreType.DMA,
            pltpu.SemaphoreType.DMA,
            pltpu.SemaphoreType.DMA,
        ],
    )(_sc_gather_body)


def kernel(z_e, codebook):
    wn, k3, loss = _dist_call(z_e, codebook)
    k = k3.reshape(NN)
    zq, util16 = _make_sc_gather()(wn, k)
    return (zq, k, loss.reshape(()), util16[0])


# speedup vs baseline: 1.0229x; 1.0229x over previous
import functools

import jax
import jax.numpy as jnp
from jax import lax
from jax.experimental import pallas as pl
from jax.experimental.pallas import tpu as pltpu
from jax.experimental.pallas import tpu_sc as plsc

KC = 8192
DD = 256
NN = 16384
BETA = 0.25

TN = 2048
TK = 4096
NT = NN // TN
NK = KC // TK
NR8 = TK // 8


def _dist_body(z_ref, cb_ref, wn_out, k_out, loss_out,
               zn2_s, w2_s, m8_s, b8_s, loss_sm):
    t = pl.program_id(0)
    kb = pl.program_id(1)

    @pl.when(t == 0)
    def _():
        wb = cb_ref[pl.ds(kb * TK, TK), :]
        nrm = jnp.sqrt(jnp.sum(wb * wb, axis=1, keepdims=True))
        wn = wb / jnp.clip(nrm, 1e-12, None)
        wn_out[pl.ds(kb * TK, TK), :] = wn
        w2_s[pl.ds(kb * TK, TK), :] = jnp.sum(wn * wn, axis=1, keepdims=True)

    @pl.when(kb == 0)
    def _():
        zb = z_ref[...]
        nrm = jnp.sqrt(jnp.sum(zb * zb, axis=1, keepdims=True))
        zn = zb / jnp.clip(nrm, 1e-12, None)
        zn2_s[...] = -2.0 * zn
        m8_s[...] = jnp.full((8, TN), jnp.inf, jnp.float32)
        b8_s[...] = jnp.zeros((8, TN), jnp.int32)
        part = jnp.sum(zn * zn)
        loss_sm[0] = jnp.where(t == 0, part, loss_sm[0] + part)

    wn = wn_out[pl.ds(kb * TK, TK), :]
    s = lax.dot_general(wn, zn2_s[...], (((1,), (1,)), ((), ())),
                        preferred_element_type=jnp.float32)
    d = s + w2_s[pl.ds(kb * TK, TK), :]
    d3 = d.reshape(NR8, 8, TN)
    m8 = m8_s[...]
    b8 = b8_s[...]
    for r in range(NR8):
        v = d3[r]
        pred = v < m8
        m8 = jnp.where(pred, v, m8)
        b8 = jnp.where(pred, jnp.int32(kb * NR8 + r), b8)
    m8_s[...] = m8
    b8_s[...] = b8

    @pl.when(kb == NK - 1)
    def _():
        cidx = b8 * 8 + lax.broadcasted_iota(jnp.int32, (8, TN), 0)
        mwin = jnp.min(m8, axis=0, keepdims=True)
        cand = jnp.where(m8 == mwin, cidx, KC)
        k_out[0, 0, :] = jnp.min(cand, axis=0)
        loss_sm[0] = loss_sm[0] + jnp.sum(mwin)

        @pl.when(t == NT - 1)
        def _():
            loss_out[...] = (BETA / float(NN * DD)) * loss_sm[0] * jnp.ones(
                (1, 1), jnp.float32)


_dist_call = pl.pallas_call(
    _dist_body,
    grid=(NT, NK),
    in_specs=[
        pl.BlockSpec((TN, DD), lambda t, kb: (t, 0)),
        pl.BlockSpec((KC, DD), lambda t, kb: (0, 0)),
    ],
    out_specs=[
        pl.BlockSpec((KC, DD), lambda t, kb: (0, 0)),
        pl.BlockSpec((1, 1, TN), lambda t, kb: (t, 0, 0)),
        pl.BlockSpec((1, 1), lambda t, kb: (0, 0)),
    ],
    out_shape=[
        jax.ShapeDtypeStruct((KC, DD), jnp.float32),
        jax.ShapeDtypeStruct((NT, 1, TN), jnp.int32),
        jax.ShapeDtypeStruct((1, 1), jnp.float32),
    ],
    scratch_shapes=[
        pltpu.VMEM((TN, DD), jnp.float32),
        pltpu.VMEM((KC, 1), jnp.float32),
        pltpu.VMEM((8, TN), jnp.float32),
        pltpu.VMEM((8, TN), jnp.int32),
        pltpu.SMEM((1,), jnp.float32),
    ],
)


CH = 128
GPW0 = 384
GPW1 = 640
NCH0 = GPW0 // CH
NCH1 = GPW1 // CH
SPT = NN // 16
NSC = SPT // CH
TSL = KC // 16


def _sc_gather_body(wn_hbm, k_hbm, zq_hbm, util_hbm,
                    idx_v, rows_v, sidx_v, zero_v, ones_v, cnt_v, util_v,
                    cnt_sm, table_sh, g0, g1, st0, st1):
    c = lax.axis_index("c")
    s = lax.axis_index("s")

    def pipeline(base, nch):
        pltpu.sync_copy(k_hbm.at[pl.ds(base, nch * CH)],
                        idx_v.at[pl.ds(0, nch * CH)])
        gs = [g0, g1]
        ss = [st0, st1]
        gh = {}
        sh = {}
        for ch in range(nch):
            b = ch % 2
            if ch >= 2:
                sh[ch - 2].wait()
            gh[ch] = pltpu.async_copy(
                wn_hbm.at[idx_v.at[pl.ds(ch * CH, CH)]],
                rows_v.at[b], gs[b])
            if ch >= 1:
                gh[ch - 1].wait()
                sh[ch - 1] = pltpu.async_copy(
                    rows_v.at[1 - b],
                    zq_hbm.at[pl.ds(base + (ch - 1) * CH, CH)], ss[1 - b])
        gh[nch - 1].wait()
        sh[nch - 1] = pltpu.async_copy(
            rows_v.at[(nch - 1) % 2],
            zq_hbm.at[pl.ds(base + (nch - 1) * CH, CH)], ss[(nch - 1) % 2])
        if nch >= 2:
            sh[nch - 2].wait()
        sh[nch - 1].wait()

    @pl.when(c == 0)
    def _():
        for i in range(TSL // 16):
            zero_v[pl.ds(i * 16, 16)] = jnp.zeros((16,), jnp.float32)
        pltpu.sync_copy(zero_v, table_sh.at[pl.ds(s * TSL, TSL)])

        @pl.when(s == 0)
        def _():
            cnt_sm[0] = jnp.int32(0)

        plsc.subcore_barrier()

        pipeline(s * GPW0, NCH0)

        for i in range(CH // 16):
            ones_v[pl.ds(i * 16, 16)] = jnp.ones((16,), jnp.float32)
        for j in range(NSC):
            pltpu.sync_copy(k_hbm.at[pl.ds(s * SPT + j * CH, CH)],
                            sidx_v.at[j])
        for j in range(NSC):
            pltpu.sync_copy(ones_v, table_sh.at[sidx_v.at[j]], add=True)
        plsc.subcore_barrier()

        pltpu.sync_copy(table_sh.at[pl.ds(s * TSL, TSL)],
                        cnt_v.at[pl.ds(0, TSL)])

        def body(i, acc):
            v = cnt_v[pl.ds(i * 16, 16)]
            return acc + jnp.where(v > 0.0, jnp.int32(1), jnp.int32(0))

        acc = lax.fori_loop(0, TSL // 16, body, jnp.zeros((16,), jnp.int32))
        my_cnt = acc[0]
        for i in range(1, 16):
            my_cnt = my_cnt + acc[i]
        plsc.fetch_and_add(cnt_sm.at[0], my_cnt, subcore_id=0)
        plsc.subcore_barrier()

        @pl.when(s == 0)
        def _():
            tot = cnt_sm[0].astype(jnp.float32)
            util_v[...] = jnp.full((16,), tot * (1.0 / float(KC)),
                                   jnp.float32)
            pltpu.sync_copy(util_v, util_hbm)

    @pl.when(c == 1)
    def _():
        pipeline(16 * GPW0 + s * GPW1, NCH1)


@functools.cache
def _make_sc_gather():
    mesh = plsc.VectorSubcoreMesh(core_axis_name="c", subcore_axis_name="s")
    return functools.partial(
        pl.kernel, mesh=mesh,
        out_type=[
            jax.ShapeDtypeStruct((NN, DD), jnp.float32),
            jax.ShapeDtypeStruct((16,), jnp.float32),
        ],
        scratch_types=[
            pltpu.VMEM((GPW1,), jnp.int32),
            pltpu.VMEM((2, CH, DD), jnp.float32),
            pltpu.VMEM((NSC, CH), jnp.int32),
            pltpu.VMEM((TSL,), jnp.float32),
            pltpu.VMEM((CH,), jnp.float32),
            pltpu.VMEM((KC // 16,), jnp.float32),
            pltpu.VMEM((16,), jnp.float32),
            pltpu.SMEM((1,), jnp.int32),
            pltpu.VMEM_SHARED((KC,), jnp.float32),
            pltpu.SemaphoreType.DMA,
            pltpu.SemaphoreType.DMA,
            pltpu.SemaphoreType.DMA,
            pltpu.SemaphoreType.DMA,
        ],
    )(_sc_gather_body)


def kernel(z_e, codebook):
    wn, k3, loss = _dist_call(z_e, codebook)
    k = k3.reshape(NN)
    zq, util16 = _make_sc_gather()(wn, k)
    return (zq, k, loss.reshape(()), util16[0])
